# baseline (speedup 1.0000x reference)
import jax
import jax.numpy as jnp
from jax import lax
from jax.experimental import pallas as pl
from jax.experimental.pallas import tpu as pltpu
from jax.experimental.pallas import tpu_sc as plsc

VOCAB = 100000
MAX_POSITION = 2048
EMBED = 1024
BATCH = 4
SEQ = 2048

NC, NS = 2, 16
NW = NC * NS
PPW = SEQ // NW
HALF = PPW // 2
QROWS = 4
NSTEP = (BATCH * SEQ) // NW // QROWS
LANES = 16
VECS = EMBED // LANES
NBUF = 4
NROUND = NSTEP // NBUF


def _embed_body(ids_hbm, tok_hbm, pos_hbm, out_hbm,
                idx_v, pos_v, b0, b1, b2, b3,
                gs0, gs1, gs2, gs3, ws0, ws1, ws2, ws3, psem):
    wid = lax.axis_index("s") * NC + lax.axis_index("c")
    pstart = wid * PPW
    bufs = (b0, b1, b2, b3)
    gsems = (gs0, gs1, gs2, gs3)
    wsems = (ws0, ws1, ws2, ws3)
    pltpu.sync_copy(ids_hbm.at[wid], idx_v)
    pltpu.async_copy(pos_hbm.at[pl.ds(pstart, HALF)], pos_v, psem)

    def sparams(s):
        h = s // 32
        b = (s % 32) // 8
        q = s % 8
        poff = h * HALF + q * QROWS
        return b, poff, q * QROWS

    def issue_gather(s, slot):
        b, poff, _ = sparams(s)
        return pltpu.async_copy(
            tok_hbm.at[idx_v.at[b, pl.ds(poff, QROWS)]],
            bufs[slot], gsems[slot])

    for s in range(NBUF - 1):
        issue_gather(s, s)

    def round_body(t, _):
        for j in range(NBUF):
            s = t * NBUF + j
            b, poff, prow = sparams(s)

            pltpu.make_async_copy(
                tok_hbm.at[idx_v.at[b, pl.ds(poff, QROWS)]],
                bufs[j], gsems[j]).wait()

            @pl.when((s == 0) | (s == NSTEP // 2))
            def _():
                pltpu.make_async_copy(
                    pos_hbm.at[pl.ds(pstart, HALF)], pos_v, psem).wait()

            @plsc.parallel_loop(0, QROWS, 1)
            def add_row(r, j=j, prow=prow):
                for v in range(VECS):
                    col = v * LANES
                    plsc.addupdate(bufs[j].at[r, pl.ds(col, LANES)],
                                   pos_v[prow + r, pl.ds(col, LANES)])

            @pl.when(s == NSTEP // 2 - 1)
            def _():
                pltpu.async_copy(
                    pos_hbm.at[pl.ds(pstart + HALF, HALF)], pos_v, psem)

            pltpu.async_copy(
                bufs[j], out_hbm.at[pl.ds(b * SEQ + pstart + poff, QROWS)],
                wsems[j])

            nslot = (j + NBUF - 1) % NBUF
            @pl.when(s >= 1)
            def _():
                pltpu.make_async_copy(
                    bufs[nslot], out_hbm.at[pl.ds(0, QROWS)],
                    wsems[nslot]).wait()

            @pl.when(s + NBUF - 1 < NSTEP)
            def _():
                issue_gather(s + NBUF - 1, nslot)
        return 0

    lax.fori_loop(0, NROUND, round_body, 0)
    pltpu.make_async_copy(
        bufs[NBUF - 1], out_hbm.at[pl.ds(0, QROWS)], wsems[NBUF - 1]).wait()


@jax.jit
def _embed(ids3, token_table, position_table):
    mesh = plsc.VectorSubcoreMesh(core_axis_name="c", subcore_axis_name="s")
    k = pl.kernel(
        _embed_body,
        out_type=jax.ShapeDtypeStruct((BATCH * SEQ, EMBED), jnp.float32),
        mesh=mesh,
        scratch_types=[
            pltpu.VMEM((BATCH, PPW), jnp.int32),
            pltpu.VMEM((HALF, EMBED), jnp.float32),
            pltpu.VMEM((QROWS, EMBED), jnp.float32),
            pltpu.VMEM((QROWS, EMBED), jnp.float32),
            pltpu.VMEM((QROWS, EMBED), jnp.float32),
            pltpu.VMEM((QROWS, EMBED), jnp.float32),
            pltpu.SemaphoreType.DMA,
            pltpu.SemaphoreType.DMA,
            pltpu.SemaphoreType.DMA,
            pltpu.SemaphoreType.DMA,
            pltpu.SemaphoreType.DMA,
            pltpu.SemaphoreType.DMA,
            pltpu.SemaphoreType.DMA,
            pltpu.SemaphoreType.DMA,
            pltpu.SemaphoreType.DMA,
        ],
    )
    return k(ids3, token_table, position_table)


def kernel(input_ids, token_table, position_table):
    ids3 = jnp.transpose(
        input_ids.astype(jnp.int32).reshape(BATCH, NW, PPW), (1, 0, 2)
    )
    out = _embed(ids3, token_table, position_table)
    return out.reshape(BATCH, SEQ, EMBED)


# speedup vs baseline: 1.3325x; 1.3325x over previous
import jax
import jax.numpy as jnp
from jax import lax
from jax.experimental import pallas as pl
from jax.experimental.pallas import tpu as pltpu
from jax.experimental.pallas import tpu_sc as plsc

VOCAB = 100000
MAX_POSITION = 2048
EMBED = 1024
BATCH = 4
SEQ = 2048

NC, NS = 2, 16
NW = NC * NS
PPW = SEQ // NW
HALF = PPW // 2
QROWS = 8
NSTEP = (BATCH * SEQ) // NW // QROWS
LANES = 16
VECS = EMBED // LANES
NBUF = 8
NROUND = NSTEP // NBUF


def _embed_body(ids_hbm, tok_hbm, pos_hbm, out_hbm,
                idx_v, pos_v, b0, b1, b2, b3, b4, b5, b6, b7,
                gs0, gs1, gs2, gs3, gs4, gs5, gs6, gs7,
                ws0, ws1, ws2, ws3, ws4, ws5, ws6, ws7, psem):
    wid = lax.axis_index("s") * NC + lax.axis_index("c")
    pstart = wid * PPW
    bufs = (b0, b1, b2, b3, b4, b5, b6, b7)
    gsems = (gs0, gs1, gs2, gs3, gs4, gs5, gs6, gs7)
    wsems = (ws0, ws1, ws2, ws3, ws4, ws5, ws6, ws7)
    pltpu.sync_copy(ids_hbm.at[wid], idx_v)
    pltpu.async_copy(pos_hbm.at[pl.ds(pstart, HALF)], pos_v, psem)

    def sparams(s):
        h = s // 16
        b = (s % 16) // 4
        q = s % 4
        poff = h * HALF + q * QROWS
        return b, poff, q * QROWS

    def issue_gather(s, slot):
        b, poff, _ = sparams(s)
        return pltpu.async_copy(
            tok_hbm.at[idx_v.at[b, pl.ds(poff, QROWS)]],
            bufs[slot], gsems[slot])

    for s in range(NBUF - 1):
        issue_gather(s, s)

    def round_body(t, _):
        for j in range(NBUF):
            s = t * NBUF + j
            b, poff, prow = sparams(s)

            pltpu.make_async_copy(
                tok_hbm.at[idx_v.at[b, pl.ds(poff, QROWS)]],
                bufs[j], gsems[j]).wait()

            @pl.when((s == 0) | (s == NSTEP // 2))
            def _():
                pltpu.make_async_copy(
                    pos_hbm.at[pl.ds(pstart, HALF)], pos_v, psem).wait()

            @plsc.parallel_loop(0, QROWS, 1)
            def add_row(r, j=j, prow=prow):
                for v in range(VECS):
                    col = v * LANES
                    plsc.addupdate(bufs[j].at[r, pl.ds(col, LANES)],
                                   pos_v[prow + r, pl.ds(col, LANES)])

            @pl.when(s == NSTEP // 2 - 1)
            def _():
                pltpu.async_copy(
                    pos_hbm.at[pl.ds(pstart + HALF, HALF)], pos_v, psem)

            pltpu.async_copy(
                bufs[j], out_hbm.at[pl.ds(b * SEQ + pstart + poff, QROWS)],
                wsems[j])

            nslot = (j + NBUF - 1) % NBUF
            @pl.when(s >= 1)
            def _():
                pltpu.make_async_copy(
                    bufs[nslot], out_hbm.at[pl.ds(0, QROWS)],
                    wsems[nslot]).wait()

            @pl.when(s + NBUF - 1 < NSTEP)
            def _():
                issue_gather(s + NBUF - 1, nslot)
        return 0

    lax.fori_loop(0, NROUND, round_body, 0)
    pltpu.make_async_copy(
        bufs[NBUF - 1], out_hbm.at[pl.ds(0, QROWS)], wsems[NBUF - 1]).wait()


@jax.jit
def _embed(ids3, token_table, position_table):
    mesh = plsc.VectorSubcoreMesh(core_axis_name="c", subcore_axis_name="s")
    k = pl.kernel(
        _embed_body,
        out_type=jax.ShapeDtypeStruct((BATCH * SEQ, EMBED), jnp.float32),
        mesh=mesh,
        scratch_types=[
            pltpu.VMEM((BATCH, PPW), jnp.int32),
            pltpu.VMEM((HALF, EMBED), jnp.float32),
            pltpu.VMEM((QROWS, EMBED), jnp.float32),
            pltpu.VMEM((QROWS, EMBED), jnp.float32),
            pltpu.VMEM((QROWS, EMBED), jnp.float32),
            pltpu.VMEM((QROWS, EMBED), jnp.float32),
            pltpu.VMEM((QROWS, EMBED), jnp.float32),
            pltpu.VMEM((QROWS, EMBED), jnp.float32),
            pltpu.VMEM((QROWS, EMBED), jnp.float32),
            pltpu.VMEM((QROWS, EMBED), jnp.float32),
            pltpu.SemaphoreType.DMA,
            pltpu.SemaphoreType.DMA,
            pltpu.SemaphoreType.DMA,
            pltpu.SemaphoreType.DMA,
            pltpu.SemaphoreType.DMA,
            pltpu.SemaphoreType.DMA,
            pltpu.SemaphoreType.DMA,
            pltpu.SemaphoreType.DMA,
            pltpu.SemaphoreType.DMA,
            pltpu.SemaphoreType.DMA,
            pltpu.SemaphoreType.DMA,
            pltpu.SemaphoreType.DMA,
            pltpu.SemaphoreType.DMA,
            pltpu.SemaphoreType.DMA,
            pltpu.SemaphoreType.DMA,
            pltpu.SemaphoreType.DMA,
            pltpu.SemaphoreType.DMA,
        ],
    )
    return k(ids3, token_table, position_table)


def kernel(input_ids, token_table, position_table):
    ids3 = jnp.transpose(
        input_ids.astype(jnp.int32).reshape(BATCH, NW, PPW), (1, 0, 2)
    )
    out = _embed(ids3, token_table, position_table)
    return out.reshape(BATCH, SEQ, EMBED)


# speedup vs baseline: 1.3866x; 1.0406x over previous
import jax
import jax.numpy as jnp
from jax import lax
from jax.experimental import pallas as pl
from jax.experimental.pallas import tpu as pltpu
from jax.experimental.pallas import tpu_sc as plsc

VOCAB = 100000
MAX_POSITION = 2048
EMBED = 1024
BATCH = 4
SEQ = 2048

NC, NS = 2, 16
NW = NC * NS
PPW = SEQ // NW
HALF = PPW // 2
QROWS = 8
NSTEP = (BATCH * SEQ) // NW // QROWS
LANES = 16
VECS = EMBED // LANES
NBUF = 4
NROUND = NSTEP // NBUF


def _embed_body(ids_hbm, tok_hbm, pos_hbm, out_hbm,
                idx_v, pos_v, b0, b1, b2, b3,
                gs0, gs1, gs2, gs3, ws0, ws1, ws2, ws3, psem):
    wid = lax.axis_index("s") * NC + lax.axis_index("c")
    pstart = wid * PPW
    bufs = (b0, b1, b2, b3)
    gsems = (gs0, gs1, gs2, gs3)
    wsems = (ws0, ws1, ws2, ws3)
    pltpu.sync_copy(ids_hbm.at[wid], idx_v)
    pltpu.async_copy(pos_hbm.at[pl.ds(pstart, HALF)], pos_v, psem)

    def sparams(s):
        h = s // 16
        b = (s % 16) // 4
        q = s % 4
        poff = h * HALF + q * QROWS
        return b, poff, q * QROWS

    def issue_gather(s, slot):
        b, poff, _ = sparams(s)
        return pltpu.async_copy(
            tok_hbm.at[idx_v.at[b, pl.ds(poff, QROWS)]],
            bufs[slot], gsems[slot])

    for s in range(NBUF - 1):
        issue_gather(s, s)

    def round_body(t, _):
        for j in range(NBUF):
            s = t * NBUF + j
            b, poff, prow = sparams(s)

            pltpu.make_async_copy(
                tok_hbm.at[idx_v.at[b, pl.ds(poff, QROWS)]],
                bufs[j], gsems[j]).wait()

            @pl.when((s == 0) | (s == NSTEP // 2))
            def _():
                pltpu.make_async_copy(
                    pos_hbm.at[pl.ds(pstart, HALF)], pos_v, psem).wait()

            @plsc.parallel_loop(0, QROWS, 1)
            def add_row(r, j=j, prow=prow):
                for v in range(VECS):
                    col = v * LANES
                    plsc.addupdate(bufs[j].at[r, pl.ds(col, LANES)],
                                   pos_v[prow + r, pl.ds(col, LANES)])

            @pl.when(s == NSTEP // 2 - 1)
            def _():
                pltpu.async_copy(
                    pos_hbm.at[pl.ds(pstart + HALF, HALF)], pos_v, psem)

            pltpu.async_copy(
                bufs[j], out_hbm.at[pl.ds(b * SEQ + pstart + poff, QROWS)],
                wsems[j])

            nslot = (j + NBUF - 1) % NBUF
            @pl.when(s >= 1)
            def _():
                pltpu.make_async_copy(
                    bufs[nslot], out_hbm.at[pl.ds(0, QROWS)],
                    wsems[nslot]).wait()

            @pl.when(s + NBUF - 1 < NSTEP)
            def _():
                issue_gather(s + NBUF - 1, nslot)
        return 0

    lax.fori_loop(0, NROUND, round_body, 0)
    pltpu.make_async_copy(
        bufs[NBUF - 1], out_hbm.at[pl.ds(0, QROWS)], wsems[NBUF - 1]).wait()


@jax.jit
def _embed(ids3, token_table, position_table):
    mesh = plsc.VectorSubcoreMesh(core_axis_name="c", subcore_axis_name="s")
    k = pl.kernel(
        _embed_body,
        out_type=jax.ShapeDtypeStruct((BATCH * SEQ, EMBED), jnp.float32),
        mesh=mesh,
        scratch_types=[
            pltpu.VMEM((BATCH, PPW), jnp.int32),
            pltpu.VMEM((HALF, EMBED), jnp.float32),
            pltpu.VMEM((QROWS, EMBED), jnp.float32),
            pltpu.VMEM((QROWS, EMBED), jnp.float32),
            pltpu.VMEM((QROWS, EMBED), jnp.float32),
            pltpu.VMEM((QROWS, EMBED), jnp.float32),
            pltpu.SemaphoreType.DMA,
            pltpu.SemaphoreType.DMA,
            pltpu.SemaphoreType.DMA,
            pltpu.SemaphoreType.DMA,
            pltpu.SemaphoreType.DMA,
            pltpu.SemaphoreType.DMA,
            pltpu.SemaphoreType.DMA,
            pltpu.SemaphoreType.DMA,
            pltpu.SemaphoreType.DMA,
        ],
    )
    return k(ids3, token_table, position_table)


def kernel(input_ids, token_table, position_table):
    ids3 = jnp.transpose(
        input_ids.astype(jnp.int32).reshape(BATCH, NW, PPW), (1, 0, 2)
    )
    out = _embed(ids3, token_table, position_table)
    return out.reshape(BATCH, SEQ, EMBED)


# speedup vs baseline: 1.3934x; 1.0049x over previous
import jax
import jax.numpy as jnp
from jax import lax
from jax.experimental import pallas as pl
from jax.experimental.pallas import tpu as pltpu
from jax.experimental.pallas import tpu_sc as plsc

VOCAB = 100000
MAX_POSITION = 2048
EMBED = 1024
BATCH = 4
SEQ = 2048

NC, NS = 2, 16
NW = NC * NS
PPW = SEQ // NW
HALF = PPW // 2
QROWS = 8
NSTEP = (BATCH * SEQ) // NW // QROWS
LANES = 16
VECS = EMBED // LANES
NBUF = 4
NROUND = NSTEP // NBUF


def _embed_body(ids_hbm, tok_hbm, pos_hbm, out_hbm,
                idx_v, pos_v, b0, b1, b2, b3,
                gs0, gs1, gs2, gs3, ws0, ws1, ws2, ws3, psem):
    wid = lax.axis_index("s") * NC + lax.axis_index("c")
    pstart = wid * PPW
    bufs = (b0, b1, b2, b3)
    gsems = (gs0, gs1, gs2, gs3)
    wsems = (ws0, ws1, ws2, ws3)
    pltpu.sync_copy(ids_hbm.at[wid], idx_v)
    pltpu.async_copy(pos_hbm.at[pl.ds(pstart, HALF)], pos_v, psem)

    def sparams(s):
        h = s // 16
        b = (s % 16) // 4
        q = s % 4
        poff = h * HALF + q * QROWS
        return b, poff, q * QROWS

    def issue_gather(s, slot):
        b, poff, _ = sparams(s)
        return pltpu.async_copy(
            tok_hbm.at[idx_v.at[b, pl.ds(poff, QROWS)]],
            bufs[slot], gsems[slot])

    for s in range(NBUF - 1):
        issue_gather(s, s)

    def round_body(t, _):
        for j in range(NBUF):
            s = t * NBUF + j
            b, poff, prow = sparams(s)

            pltpu.make_async_copy(
                tok_hbm.at[pl.ds(0, QROWS)], bufs[j], gsems[j]).wait()

            @pl.when((s == 0) | (s == NSTEP // 2))
            def _():
                pltpu.make_async_copy(
                    pos_hbm.at[pl.ds(pstart, HALF)], pos_v, psem).wait()

            @plsc.parallel_loop(0, QROWS, 1)
            def add_row(r, j=j, prow=prow):
                for v in range(VECS):
                    col = v * LANES
                    plsc.addupdate(bufs[j].at[r, pl.ds(col, LANES)],
                                   pos_v[prow + r, pl.ds(col, LANES)])

            @pl.when(s == NSTEP // 2 - 1)
            def _():
                pltpu.async_copy(
                    pos_hbm.at[pl.ds(pstart + HALF, HALF)], pos_v, psem)

            pltpu.async_copy(
                bufs[j], out_hbm.at[pl.ds(b * SEQ + pstart + poff, QROWS)],
                wsems[j])

            nslot = (j + NBUF - 1) % NBUF
            @pl.when(s >= 1)
            def _():
                pltpu.make_async_copy(
                    bufs[nslot], out_hbm.at[pl.ds(0, QROWS)],
                    wsems[nslot]).wait()

            @pl.when(s + NBUF - 1 < NSTEP)
            def _():
                issue_gather(s + NBUF - 1, nslot)
        return 0

    lax.fori_loop(0, NROUND, round_body, 0)
    pltpu.make_async_copy(
        bufs[NBUF - 1], out_hbm.at[pl.ds(0, QROWS)], wsems[NBUF - 1]).wait()


@jax.jit
def _embed(ids3, token_table, position_table):
    mesh = plsc.VectorSubcoreMesh(core_axis_name="c", subcore_axis_name="s")
    k = pl.kernel(
        _embed_body,
        out_type=jax.ShapeDtypeStruct((BATCH * SEQ, EMBED), jnp.float32),
        mesh=mesh,
        scratch_types=[
            pltpu.VMEM((BATCH, PPW), jnp.int32),
            pltpu.VMEM((HALF, EMBED), jnp.float32),
            pltpu.VMEM((QROWS, EMBED), jnp.float32),
            pltpu.VMEM((QROWS, EMBED), jnp.float32),
            pltpu.VMEM((QROWS, EMBED), jnp.float32),
            pltpu.VMEM((QROWS, EMBED), jnp.float32),
            pltpu.SemaphoreType.DMA,
            pltpu.SemaphoreType.DMA,
            pltpu.SemaphoreType.DMA,
            pltpu.SemaphoreType.DMA,
            pltpu.SemaphoreType.DMA,
            pltpu.SemaphoreType.DMA,
            pltpu.SemaphoreType.DMA,
            pltpu.SemaphoreType.DMA,
            pltpu.SemaphoreType.DMA,
        ],
    )
    return k(ids3, token_table, position_table)


def kernel(input_ids, token_table, position_table):
    ids3 = jnp.transpose(
        input_ids.astype(jnp.int32).reshape(BATCH, NW, PPW), (1, 0, 2)
    )
    out = _embed(ids3, token_table, position_table)
    return out.reshape(BATCH, SEQ, EMBED)
